# Initial kernel scaffold; baseline (speedup 1.0000x reference)
import jax
import jax.numpy as jnp
from jax.experimental import pallas as pl


def kernel(x, neighbor_indices, distancesq, W0, b0, W1, b1):
    raise NotImplementedError("write your pallas kernel here")


# speedup vs baseline: 1.1084x; 1.1084x over previous
import functools

import jax
import jax.numpy as jnp
from jax import lax
from jax.experimental import pallas as pl
from jax.experimental.pallas import tpu as pltpu
from jax.experimental.pallas import tpu_sc as plsc

_N = 10000
_K = 16
_D = 256
_H = 256
_LANES = 16
_NWORKERS = 32
_CHUNK = 8
_NP = 10240
_NPW = _NP // _NWORKERS
_NCHUNKS = _NPW // _CHUNK
_NG = _H // _LANES


def _mm_relu(a, w, b):
    m, kd = a.shape
    hd = w.shape[1]
    bm = 1024

    def body(a_ref, w_ref, b_ref, o_ref):
        acc = jnp.dot(a_ref[...], w_ref[...],
                      preferred_element_type=jnp.float32)
        o_ref[...] = jnp.maximum(acc + b_ref[...], 0.0)

    return pl.pallas_call(
        body,
        grid=(m // bm,),
        in_specs=[
            pl.BlockSpec((bm, kd), lambda i: (i, 0)),
            pl.BlockSpec((kd, hd), lambda i: (0, 0)),
            pl.BlockSpec((1, hd), lambda i: (0, 0)),
        ],
        out_specs=pl.BlockSpec((bm, hd), lambda i: (i, 0)),
        out_shape=jax.ShapeDtypeStruct((m, hd), jnp.float32),
    )(a, w, b.reshape(1, hd))


def _sc_acc_body(feat_hbm, idx_hbm, dsq_hbm, out_hbm,
                 idx_v, rows_v, dsq_v, w_v, own_v, out_v, sem):
    wid = lax.axis_index("s") * 2 + lax.axis_index("c")
    base = wid * _NPW

    def chunk(ci, carry):
        row0 = base + ci * _CHUNK
        e0 = row0 * _K
        pltpu.sync_copy(idx_hbm.at[pl.ds(e0, _CHUNK * _K)], idx_v)
        pltpu.sync_copy(dsq_hbm.at[pl.ds(e0, _CHUNK * _K)], dsq_v)
        pltpu.sync_copy(feat_hbm.at[pl.ds(row0, _CHUNK)], own_v)
        gather = pltpu.async_copy(feat_hbm.at[idx_v], rows_v, sem)
        for j in range(_CHUNK * _K // _LANES):
            sl = pl.ds(j * _LANES, _LANES)
            w_v[sl] = jnp.exp(dsq_v[sl] * -10.0)
        gather.wait()

        def node(n, ncarry):
            rbase = n * _K
            wk = [plsc.load_gather(
                      w_v, [jnp.full((_LANES,), rbase + k, jnp.int32)])
                  for k in range(_K)]
            for g in range(_NG):
                col = g * _LANES
                s = None
                mx = None
                for k in range(_K):
                    p = rows_v[rbase + k, pl.ds(col, _LANES)] * wk[k]
                    s = p if s is None else s + p
                    mx = p if mx is None else jnp.maximum(mx, p)
                own = own_v[n, pl.ds(col, _LANES)]
                out_v[n, pl.ds(col, _LANES)] = s * (1.0 / _K) - own
                out_v[n, pl.ds(_H + col, _LANES)] = mx - own
            return ncarry

        lax.fori_loop(0, _CHUNK, node, 0)
        pltpu.sync_copy(out_v, out_hbm.at[pl.ds(row0, _CHUNK)])
        return carry

    lax.fori_loop(0, _NCHUNKS, chunk, 0)


_sc_acc = functools.partial(
    pl.kernel,
    out_type=jax.ShapeDtypeStruct((_NP, 2 * _H), jnp.float32),
    mesh=plsc.VectorSubcoreMesh(core_axis_name="c", subcore_axis_name="s",
                                num_cores=2, num_subcores=16),
    compiler_params=pltpu.CompilerParams(needs_layout_passes=False),
    scratch_types=[
        pltpu.VMEM((_CHUNK * _K,), jnp.int32),
        pltpu.VMEM((_CHUNK * _K, _H), jnp.float32),
        pltpu.VMEM((_CHUNK * _K,), jnp.float32),
        pltpu.VMEM((_CHUNK * _K,), jnp.float32),
        pltpu.VMEM((_CHUNK, _H), jnp.float32),
        pltpu.VMEM((_CHUNK, 2 * _H), jnp.float32),
        pltpu.SemaphoreType.DMA,
    ],
)(_sc_acc_body)


def kernel(x, neighbor_indices, distancesq, W0, b0, W1, b1):
    xp = jnp.pad(x, ((0, _NP - _N), (0, 0)))
    idxp = jnp.pad(neighbor_indices.reshape(-1), (0, (_NP - _N) * _K))
    dsqp = jnp.pad(distancesq.reshape(-1), (0, (_NP - _N) * _K))
    f0 = _mm_relu(xp, W0, b0)
    f1 = _sc_acc(f0, idxp, dsqp)
    h1 = _mm_relu(f1, W1, b1)
    f2 = _sc_acc(h1, idxp, dsqp)
    return jnp.concatenate([f1[:_N], f2[:_N], x], axis=-1)


# speedup vs baseline: 1.5426x; 1.3918x over previous
import functools

import jax
import jax.numpy as jnp
from jax import lax
from jax.experimental import pallas as pl
from jax.experimental.pallas import tpu as pltpu
from jax.experimental.pallas import tpu_sc as plsc

_N = 10000
_K = 16
_D = 256
_H = 256
_LANES = 16
_NWORKERS = 32
_CHUNK = 8
_CK = _CHUNK * _K
_NP = 10240
_NPW = _NP // _NWORKERS
_NCHUNKS = _NPW // _CHUNK
_NG = _H // _LANES


def _mm_relu(a, w, b):
    m, kd = a.shape
    hd = w.shape[1]
    bm = 1024

    def body(a_ref, w_ref, b_ref, o_ref):
        acc = jnp.dot(a_ref[...], w_ref[...],
                      preferred_element_type=jnp.float32)
        o_ref[...] = jnp.maximum(acc + b_ref[...], 0.0)

    return pl.pallas_call(
        body,
        grid=(m // bm,),
        in_specs=[
            pl.BlockSpec((bm, kd), lambda i: (i, 0)),
            pl.BlockSpec((kd, hd), lambda i: (0, 0)),
            pl.BlockSpec((1, hd), lambda i: (0, 0)),
        ],
        out_specs=pl.BlockSpec((bm, hd), lambda i: (i, 0)),
        out_shape=jax.ShapeDtypeStruct((m, hd), jnp.float32),
    )(a, w, b.reshape(1, hd))


def _tree(vals, op):
    while len(vals) > 1:
        vals = [op(vals[i], vals[i + 1]) for i in range(0, len(vals) - 1, 2)] \
            + ([vals[-1]] if len(vals) % 2 else [])
    return vals[0]


def _sc_acc_body(feat_hbm, idx_hbm, dsq_hbm, out_hbm,
                 idx_all, w_all, rows0, rows1, own0, own1, out0, out1,
                 g0, g1, o0, o1, s0, s1):
    wid = lax.axis_index("s") * 2 + lax.axis_index("c")
    base = wid * _NPW

    def gather_start(ci, rows, sem):
        pltpu.async_copy(feat_hbm.at[idx_all.at[ci]], rows, sem)

    def gather_wait(rows, sem):
        pltpu.make_async_copy(feat_hbm.at[idx_all.at[0]], rows, sem).wait()

    def own_start(ci, own, sem):
        pltpu.async_copy(
            feat_hbm.at[pl.ds(base + ci * _CHUNK, _CHUNK)], own, sem)

    def own_wait(own, sem):
        pltpu.make_async_copy(
            feat_hbm.at[pl.ds(0, _CHUNK)], own, sem).wait()

    def store_start(ci, outv, sem):
        pltpu.async_copy(
            outv, out_hbm.at[pl.ds(base + ci * _CHUNK, _CHUNK)], sem)

    def store_wait(outv, sem):
        pltpu.make_async_copy(
            outv, out_hbm.at[pl.ds(0, _CHUNK)], sem).wait()

    pltpu.sync_copy(idx_hbm.at[wid], idx_all)
    pltpu.sync_copy(dsq_hbm.at[wid], w_all)
    gather_start(0, rows0, g0)
    gather_start(1, rows1, g1)
    own_start(0, own0, o0)
    own_start(1, own1, o1)

    def expbody(j, c):
        sl = pl.ds(j * _LANES, _LANES)
        w_all[sl] = jnp.exp(w_all[sl] * -10.0)
        return c

    lax.fori_loop(0, _NPW * _K // _LANES, expbody, 0)

    def compute(ci, rows, own, outv):
        def node(n, c):
            wrow = w_all[pl.ds((ci * _CHUNK + n) * _K, _K)]
            dnums = lax.GatherDimensionNumbers(
                offset_dims=(), collapsed_slice_dims=(0,),
                start_index_map=(0,))
            wk = [lax.gather(wrow, jnp.full((_LANES, 1), k, jnp.int32),
                             dnums, slice_sizes=(1,),
                             mode=lax.GatherScatterMode.PROMISE_IN_BOUNDS)
                  for k in range(_K)]
            rbase = n * _K
            for g in range(_NG):
                col = g * _LANES
                p = [rows[rbase + k, pl.ds(col, _LANES)] * wk[k]
                     for k in range(_K)]
                s = _tree(p, lambda a, b: a + b)
                mx = _tree(p, jnp.maximum)
                ownv = own[n, pl.ds(col, _LANES)]
                outv[n, pl.ds(col, _LANES)] = s * (1.0 / _K) - ownv
                outv[n, pl.ds(_H + col, _LANES)] = mx - ownv
            return c

        lax.fori_loop(0, _CHUNK, node, 0)

    def pair(i, c):
        ci = 2 * i
        gather_wait(rows0, g0)
        own_wait(own0, o0)

        @pl.when(i > 0)
        def _():
            store_wait(out0, s0)

        compute(ci, rows0, own0, out0)
        gather_start(ci + 2, rows0, g0)
        own_start(ci + 2, own0, o0)
        store_start(ci, out0, s0)

        gather_wait(rows1, g1)
        own_wait(own1, o1)

        @pl.when(i > 0)
        def _():
            store_wait(out1, s1)

        compute(ci + 1, rows1, own1, out1)
        gather_start(ci + 3, rows1, g1)
        own_start(ci + 3, own1, o1)
        store_start(ci + 1, out1, s1)
        return c

    lax.fori_loop(0, (_NCHUNKS - 2) // 2, pair, 0)

    gather_wait(rows0, g0)
    own_wait(own0, o0)
    store_wait(out0, s0)
    compute(_NCHUNKS - 2, rows0, own0, out0)
    store_start(_NCHUNKS - 2, out0, s0)

    gather_wait(rows1, g1)
    own_wait(own1, o1)
    store_wait(out1, s1)
    compute(_NCHUNKS - 1, rows1, own1, out1)
    store_start(_NCHUNKS - 1, out1, s1)

    store_wait(out0, s0)
    store_wait(out1, s1)


_sc_acc = functools.partial(
    pl.kernel,
    out_type=jax.ShapeDtypeStruct((_NP, 2 * _H), jnp.float32),
    mesh=plsc.VectorSubcoreMesh(core_axis_name="c", subcore_axis_name="s",
                                num_cores=2, num_subcores=16),
    compiler_params=pltpu.CompilerParams(needs_layout_passes=False),
    scratch_types=[
        pltpu.VMEM((_NCHUNKS, _CK), jnp.int32),
        pltpu.VMEM((_NPW * _K,), jnp.float32),
        pltpu.VMEM((_CK, _H), jnp.float32),
        pltpu.VMEM((_CK, _H), jnp.float32),
        pltpu.VMEM((_CHUNK, _H), jnp.float32),
        pltpu.VMEM((_CHUNK, _H), jnp.float32),
        pltpu.VMEM((_CHUNK, 2 * _H), jnp.float32),
        pltpu.VMEM((_CHUNK, 2 * _H), jnp.float32),
        pltpu.SemaphoreType.DMA,
        pltpu.SemaphoreType.DMA,
        pltpu.SemaphoreType.DMA,
        pltpu.SemaphoreType.DMA,
        pltpu.SemaphoreType.DMA,
        pltpu.SemaphoreType.DMA,
    ],
)(_sc_acc_body)


def kernel(x, neighbor_indices, distancesq, W0, b0, W1, b1):
    xp = jnp.pad(x, ((0, _NP - _N), (0, 0)))
    idxp = jnp.pad(neighbor_indices.reshape(-1), (0, (_NP - _N) * _K))
    dsqp = jnp.pad(distancesq.reshape(-1), (0, (_NP - _N) * _K))
    idxw = idxp.reshape(_NWORKERS, _NCHUNKS, _CK)
    dsqw = dsqp.reshape(_NWORKERS, _NPW * _K)
    f0 = _mm_relu(xp, W0, b0)
    f1 = _sc_acc(f0, idxw, dsqw)
    h1 = _mm_relu(f1, W1, b1)
    f2 = _sc_acc(h1, idxw, dsqw)
    return jnp.concatenate([f1[:_N], f2[:_N], x], axis=-1)


# speedup vs baseline: 1.6459x; 1.0669x over previous
import functools

import jax
import jax.numpy as jnp
from jax import lax
from jax.experimental import pallas as pl
from jax.experimental.pallas import tpu as pltpu
from jax.experimental.pallas import tpu_sc as plsc

_N = 10000
_K = 16
_D = 256
_H = 256
_LANES = 16
_NTILES = 16
_CHUNK = 8
_CK = _CHUNK * _K
_NP = 10240
_NPT0 = 448
_NPT1 = 192
_NC0 = _NPT0 // _CHUNK
_NC1 = _NPT1 // _CHUNK
_MAXC = max(_NC0, _NC1)
_NE = _NP * _K + (_NPT0 - _NPT1) * _K
_NG = _H // _LANES


def _mm_relu(a, w, b):
    m, kd = a.shape
    hd = w.shape[1]
    bm = 1024

    def body(a_ref, w_ref, b_ref, o_ref):
        acc = jnp.dot(a_ref[...], w_ref[...],
                      preferred_element_type=jnp.float32)
        o_ref[...] = jnp.maximum(acc + b_ref[...], 0.0)

    return pl.pallas_call(
        body,
        grid=(m // bm,),
        in_specs=[
            pl.BlockSpec((bm, kd), lambda i: (i, 0)),
            pl.BlockSpec((kd, hd), lambda i: (0, 0)),
            pl.BlockSpec((1, hd), lambda i: (0, 0)),
        ],
        out_specs=pl.BlockSpec((bm, hd), lambda i: (i, 0)),
        out_shape=jax.ShapeDtypeStruct((m, hd), jnp.float32),
    )(a, w, b.reshape(1, hd))


def _tree(vals, op):
    while len(vals) > 1:
        vals = [op(vals[i], vals[i + 1]) for i in range(0, len(vals) - 1, 2)] \
            + ([vals[-1]] if len(vals) % 2 else [])
    return vals[0]


def _sc_acc_body(feat_hbm, idx_hbm, dsq_hbm, out_hbm,
                 idx_all, w_all, rows0, rows1, own0, own1, out0, out1,
                 g0, g1, o0, o1, s0, s1):
    cid = lax.axis_index("c")
    sid = lax.axis_index("s")
    base = jnp.where(cid == 0, sid * _NPT0,
                     _NTILES * _NPT0 + sid * _NPT1)
    base = pl.multiple_of(base, 64)
    nchunks = jnp.where(cid == 0, _NC0, _NC1)

    def gather_start(ci, rows, sem):
        pltpu.async_copy(feat_hbm.at[idx_all.at[ci]], rows, sem)

    def gather_wait(rows, sem):
        pltpu.make_async_copy(feat_hbm.at[idx_all.at[0]], rows, sem).wait()

    def row0(ci):
        return pl.multiple_of(base + ci * _CHUNK, _CHUNK)

    def own_start(ci, own, sem):
        pltpu.async_copy(
            feat_hbm.at[pl.ds(row0(ci), _CHUNK)], own, sem)

    def own_wait(own, sem):
        pltpu.make_async_copy(
            feat_hbm.at[pl.ds(0, _CHUNK)], own, sem).wait()

    def store_start(ci, outv, sem):
        pltpu.async_copy(
            outv, out_hbm.at[pl.ds(row0(ci), _CHUNK)], sem)

    def store_wait(outv, sem):
        pltpu.make_async_copy(
            outv, out_hbm.at[pl.ds(0, _CHUNK)], sem).wait()

    pltpu.sync_copy(
        idx_hbm.at[pl.ds(pl.multiple_of(base // _CHUNK, 8), _MAXC)], idx_all)
    pltpu.sync_copy(dsq_hbm.at[pl.ds(base * _K, _NPT0 * _K)], w_all)
    gather_start(0, rows0, g0)
    gather_start(1, rows1, g1)
    own_start(0, own0, o0)
    own_start(1, own1, o1)

    def expbody(j, c):
        sl = pl.ds(j * _LANES, _LANES)
        w_all[sl] = jnp.exp(w_all[sl] * -10.0)
        return c

    lax.fori_loop(0, _NPT0 * _K // _LANES, expbody, 0)

    def compute(ci, rows, own, outv):
        def node(n, c):
            wrow = w_all[pl.ds((ci * _CHUNK + n) * _K, _K)]
            dnums = lax.GatherDimensionNumbers(
                offset_dims=(), collapsed_slice_dims=(0,),
                start_index_map=(0,))
            wk = [lax.gather(wrow, jnp.full((_LANES, 1), k, jnp.int32),
                             dnums, slice_sizes=(1,),
                             mode=lax.GatherScatterMode.PROMISE_IN_BOUNDS)
                  for k in range(_K)]
            rbase = n * _K
            for g in range(_NG):
                col = g * _LANES
                p = [rows[rbase + k, pl.ds(col, _LANES)] * wk[k]
                     for k in range(_K)]
                s = _tree(p, lambda a, b: a + b)
                mx = _tree(p, jnp.maximum)
                ownv = own[n, pl.ds(col, _LANES)]
                outv[n, pl.ds(col, _LANES)] = s * (1.0 / _K) - ownv
                outv[n, pl.ds(_H + col, _LANES)] = mx - ownv
            return c

        lax.fori_loop(0, _CHUNK, node, 0)

    def pair(i, c):
        ci = 2 * i
        gather_wait(rows0, g0)
        own_wait(own0, o0)

        @pl.when(i > 0)
        def _():
            store_wait(out0, s0)

        compute(ci, rows0, own0, out0)
        gather_start(ci + 2, rows0, g0)
        own_start(ci + 2, own0, o0)
        store_start(ci, out0, s0)

        gather_wait(rows1, g1)
        own_wait(own1, o1)

        @pl.when(i > 0)
        def _():
            store_wait(out1, s1)

        compute(ci + 1, rows1, own1, out1)
        gather_start(ci + 3, rows1, g1)
        own_start(ci + 3, own1, o1)
        store_start(ci + 1, out1, s1)
        return c

    lax.fori_loop(0, (nchunks - 2) // 2, pair, 0)

    gather_wait(rows0, g0)
    own_wait(own0, o0)
    store_wait(out0, s0)
    compute(nchunks - 2, rows0, own0, out0)
    store_start(nchunks - 2, out0, s0)

    gather_wait(rows1, g1)
    own_wait(own1, o1)
    store_wait(out1, s1)
    compute(nchunks - 1, rows1, own1, out1)
    store_start(nchunks - 1, out1, s1)

    store_wait(out0, s0)
    store_wait(out1, s1)


_sc_acc = functools.partial(
    pl.kernel,
    out_type=jax.ShapeDtypeStruct((_NP, 2 * _H), jnp.float32),
    mesh=plsc.VectorSubcoreMesh(core_axis_name="c", subcore_axis_name="s",
                                num_cores=2, num_subcores=16),
    compiler_params=pltpu.CompilerParams(needs_layout_passes=False),
    scratch_types=[
        pltpu.VMEM((_MAXC, _CK), jnp.int32),
        pltpu.VMEM((_NPT0 * _K,), jnp.float32),
        pltpu.VMEM((_CK, _H), jnp.float32),
        pltpu.VMEM((_CK, _H), jnp.float32),
        pltpu.VMEM((_CHUNK, _H), jnp.float32),
        pltpu.VMEM((_CHUNK, _H), jnp.float32),
        pltpu.VMEM((_CHUNK, 2 * _H), jnp.float32),
        pltpu.VMEM((_CHUNK, 2 * _H), jnp.float32),
        pltpu.SemaphoreType.DMA,
        pltpu.SemaphoreType.DMA,
        pltpu.SemaphoreType.DMA,
        pltpu.SemaphoreType.DMA,
        pltpu.SemaphoreType.DMA,
        pltpu.SemaphoreType.DMA,
    ],
)(_sc_acc_body)


def kernel(x, neighbor_indices, distancesq, W0, b0, W1, b1):
    xp = jnp.pad(x, ((0, _NP - _N), (0, 0)))
    idxp = jnp.pad(neighbor_indices.reshape(-1), (0, _NE - _N * _K))
    dsqp = jnp.pad(distancesq.reshape(-1), (0, _NE - _N * _K))
    idxw = idxp.reshape(_NE // _CK, _CK)
    f0 = _mm_relu(xp, W0, b0)
    f1 = _sc_acc(f0, idxw, dsqp)
    h1 = _mm_relu(f1, W1, b1)
    f2 = _sc_acc(h1, idxw, dsqp)
    return jnp.concatenate([f1[:_N], f2[:_N], x], axis=-1)


# speedup vs baseline: 1.8451x; 1.1210x over previous
import functools

import jax
import jax.numpy as jnp
from jax import lax
from jax.experimental import pallas as pl
from jax.experimental.pallas import tpu as pltpu
from jax.experimental.pallas import tpu_sc as plsc

_N = 10000
_K = 16
_D = 256
_H = 256
_LANES = 16
_NTILES = 16
_CHUNK = 4
_CK = _CHUNK * _K
_NBUF = 4
_NP = 10240
_NPT0 = 448
_NPT1 = 192
_NC0 = _NPT0 // _CHUNK
_NC1 = _NPT1 // _CHUNK
_NE = _NP * _K + (_NPT0 - _NPT1) * _K
_NG = _H // _LANES


def _mm_relu(a, w, b):
    m, kd = a.shape
    hd = w.shape[1]
    bm = 1024

    def body(a_ref, w_ref, b_ref, o_ref):
        acc = jnp.dot(a_ref[...], w_ref[...],
                      preferred_element_type=jnp.float32)
        o_ref[...] = jnp.maximum(acc + b_ref[...], 0.0)

    return pl.pallas_call(
        body,
        grid=(m // bm,),
        in_specs=[
            pl.BlockSpec((bm, kd), lambda i: (i, 0)),
            pl.BlockSpec((kd, hd), lambda i: (0, 0)),
            pl.BlockSpec((1, hd), lambda i: (0, 0)),
        ],
        out_specs=pl.BlockSpec((bm, hd), lambda i: (i, 0)),
        out_shape=jax.ShapeDtypeStruct((m, hd), jnp.float32),
    )(a, w, b.reshape(1, hd))


def _tree(vals, op):
    while len(vals) > 1:
        vals = [op(vals[i], vals[i + 1]) for i in range(0, len(vals) - 1, 2)] \
            + ([vals[-1]] if len(vals) % 2 else [])
    return vals[0]


def _sc_acc_body(feat_hbm, idx_hbm, dsq_hbm, out_hbm, idx_all, w_all,
                 r0, r1, r2, r3, n0, n1, n2, n3, u0, u1, u2, u3,
                 gs0, gs1, gs2, gs3, os0, os1, os2, os3,
                 ss0, ss1, ss2, ss3):
    rows = [r0, r1, r2, r3]
    own = [n0, n1, n2, n3]
    out = [u0, u1, u2, u3]
    gsem = [gs0, gs1, gs2, gs3]
    osem = [os0, os1, os2, os3]
    ssem = [ss0, ss1, ss2, ss3]

    cid = lax.axis_index("c")
    sid = lax.axis_index("s")
    base = jnp.where(cid == 0, sid * _NPT0,
                     _NTILES * _NPT0 + sid * _NPT1)
    base = pl.multiple_of(base, 64)
    nchunks = jnp.where(cid == 0, _NC0, _NC1)

    def gather_start(ci, b):
        pltpu.async_copy(
            feat_hbm.at[idx_all.at[pl.ds(ci * _CK, _CK)]], rows[b], gsem[b])

    def gather_wait(b):
        pltpu.make_async_copy(
            feat_hbm.at[idx_all.at[pl.ds(0, _CK)]], rows[b], gsem[b]).wait()

    def row0(ci):
        return pl.multiple_of(base + ci * _CHUNK, _CHUNK)

    def own_start(ci, b):
        pltpu.async_copy(
            feat_hbm.at[pl.ds(row0(ci), _CHUNK)], own[b], osem[b])

    def own_wait(b):
        pltpu.make_async_copy(
            feat_hbm.at[pl.ds(0, _CHUNK)], own[b], osem[b]).wait()

    def store_start(ci, b):
        pltpu.async_copy(
            out[b], out_hbm.at[pl.ds(row0(ci), _CHUNK)], ssem[b])

    def store_wait(b):
        pltpu.make_async_copy(
            out[b], out_hbm.at[pl.ds(0, _CHUNK)], ssem[b]).wait()

    pltpu.sync_copy(dsq_hbm.at[pl.ds(base * _K, _NPT0 * _K)], w_all)
    pltpu.sync_copy(idx_hbm.at[pl.ds(base * _K, _NPT0 * _K)], idx_all)
    for b in range(_NBUF):
        gather_start(b, b)
        own_start(b, b)

    def expbody(j, c):
        sl = pl.ds(j * _LANES, _LANES)
        w_all[sl] = jnp.exp(w_all[sl] * -10.0)
        return c

    lax.fori_loop(0, _NPT0 * _K // _LANES, expbody, 0)

    def compute(ci, b):
        rbuf = rows[b]
        obuf = own[b]
        ubuf = out[b]

        def node(n, c):
            wrow = w_all[pl.ds((ci * _CHUNK + n) * _K, _K)]
            dnums = lax.GatherDimensionNumbers(
                offset_dims=(), collapsed_slice_dims=(0,),
                start_index_map=(0,))
            wk = [lax.gather(wrow, jnp.full((_LANES, 1), k, jnp.int32),
                             dnums, slice_sizes=(1,),
                             mode=lax.GatherScatterMode.PROMISE_IN_BOUNDS)
                  for k in range(_K)]
            rbase = n * _K
            for g in range(_NG):
                col = g * _LANES
                p = [rbuf[rbase + k, pl.ds(col, _LANES)] * wk[k]
                     for k in range(_K)]
                s = _tree(p, lambda a, b_: a + b_)
                mx = _tree(p, jnp.maximum)
                ownv = obuf[n, pl.ds(col, _LANES)]
                ubuf[n, pl.ds(col, _LANES)] = s * (1.0 / _K) - ownv
                ubuf[n, pl.ds(_H + col, _LANES)] = mx - ownv
            return c

        lax.fori_loop(0, _CHUNK, node, 0)

    def group(i, c):
        for b in range(_NBUF):
            cch = i * _NBUF + b
            gather_wait(b)
            own_wait(b)

            @pl.when(i > 0)
            def _():
                store_wait(b)

            compute(cch, b)

            @pl.when(cch + _NBUF < nchunks)
            def _():
                gather_start(cch + _NBUF, b)
                own_start(cch + _NBUF, b)

            store_start(cch, b)
        return c

    lax.fori_loop(0, nchunks // _NBUF, group, 0)

    for b in range(_NBUF):
        store_wait(b)


_sc_acc = functools.partial(
    pl.kernel,
    out_type=jax.ShapeDtypeStruct((_NP, 2 * _H), jnp.float32),
    mesh=plsc.VectorSubcoreMesh(core_axis_name="c", subcore_axis_name="s",
                                num_cores=2, num_subcores=16),
    compiler_params=pltpu.CompilerParams(needs_layout_passes=False),
    scratch_types=(
        [pltpu.VMEM((_NPT0 * _K,), jnp.int32),
         pltpu.VMEM((_NPT0 * _K,), jnp.float32)]
        + [pltpu.VMEM((_CK, _H), jnp.float32) for _ in range(_NBUF)]
        + [pltpu.VMEM((_CHUNK, _H), jnp.float32) for _ in range(_NBUF)]
        + [pltpu.VMEM((_CHUNK, 2 * _H), jnp.float32) for _ in range(_NBUF)]
        + [pltpu.SemaphoreType.DMA for _ in range(3 * _NBUF)]
    ),
)(_sc_acc_body)


def kernel(x, neighbor_indices, distancesq, W0, b0, W1, b1):
    xp = jnp.pad(x, ((0, _NP - _N), (0, 0)))
    idxp = jnp.pad(neighbor_indices.reshape(-1), (0, _NE - _N * _K))
    dsqp = jnp.pad(distancesq.reshape(-1), (0, _NE - _N * _K))
    f0 = _mm_relu(xp, W0, b0)
    f1 = _sc_acc(f0, idxp, dsqp)
    h1 = _mm_relu(f1, W1, b1)
    f2 = _sc_acc(h1, idxp, dsqp)
    return jnp.concatenate([f1[:_N], f2[:_N], x], axis=-1)
